# baseline (speedup 1.0000x reference)
import functools

import jax
import jax.numpy as jnp
from jax import lax
from jax.experimental import pallas as pl
from jax.experimental.pallas import tpu as pltpu
from jax.experimental.pallas import tpu_sc as plsc

N = 10000
NEMB = 128


_CHUNKS = (128, 128, 64)
_BPW = sum(_CHUNKS)
_TAIL_ROWS = N - 31 * _BPW


def _make_sc_gather(num_feat):
    info = plsc.get_sparse_core_info()
    nw = info.num_cores * info.num_subcores
    assert nw * _BPW >= N and (nw - 1) * _BPW + _TAIL_ROWS == N
    n_chunks = len(_CHUNKS)
    offs = [sum(_CHUNKS[:j]) for j in range(n_chunks)]
    mesh = plsc.VectorSubcoreMesh(core_axis_name="c", subcore_axis_name="s")

    @functools.partial(
        pl.kernel,
        mesh=mesh,
        out_type=jax.ShapeDtypeStruct((N, NEMB), jnp.float32),
        scratch_types=[
            pltpu.VMEM((_BPW,), jnp.int32),
            pltpu.VMEM((_BPW, NEMB), jnp.float32),
            pltpu.VMEM((_TAIL_ROWS,), jnp.int32),
            pltpu.VMEM((_TAIL_ROWS, NEMB), jnp.float32),
        ] + [pltpu.SemaphoreType.DMA] * (n_chunks + 1),
    )
    def gather_kernel(table_hbm, idx_hbm, out_hbm, idx_v, rows_v,
                      idx_t, rows_t, *sems):
        wid = lax.axis_index("s") * info.num_cores + lax.axis_index("c")
        base = wid * _BPW

        @pl.when(wid < nw - 1)
        def _full():
            pltpu.sync_copy(idx_hbm.at[pl.ds(base, _BPW)], idx_v)
            gathers = [
                pltpu.async_copy(
                    table_hbm.at[idx_v.at[pl.ds(offs[j], _CHUNKS[j])]],
                    rows_v.at[pl.ds(offs[j], _CHUNKS[j])], sems[j])
                for j in range(n_chunks)
            ]
            for g in gathers:
                g.wait()
            pltpu.async_copy(rows_v, out_hbm.at[pl.ds(base, _BPW)],
                             sems[n_chunks]).wait()

        @pl.when(wid == nw - 1)
        def _tail():
            pltpu.sync_copy(idx_hbm.at[pl.ds(base, _TAIL_ROWS)], idx_t)
            pltpu.async_copy(table_hbm.at[idx_t], rows_t, sems[0]).wait()
            pltpu.async_copy(rows_t, out_hbm.at[pl.ds(base, _TAIL_ROWS)],
                             sems[n_chunks]).wait()

    return gather_kernel


_BM = 400


def _dot16(a, b):
    return jnp.dot(a.astype(jnp.bfloat16), b.astype(jnp.bfloat16),
                   preferred_element_type=jnp.float32)


def _gcn_body(adj_ref, e_ref, w1_ref, b1_ref, w2_ref, lw1_ref, lb1_ref,
              lw2_ref, lb2_ref, b2_ref, x_ref, s16_s, t16_s):
    p = pl.program_id(0)
    m = pl.program_id(1)

    @pl.when(jnp.logical_and(p == 0, m == 0))
    def _init():
        s16_s[...] = _dot16(e_ref[...], w1_ref[...]).astype(jnp.bfloat16)

    @pl.when(p == 0)
    def _phase0():
        a16 = adj_ref[...].astype(jnp.bfloat16)
        h1 = jnp.dot(a16, s16_s[...],
                     preferred_element_type=jnp.float32) + b1_ref[...]
        r = jnp.maximum(h1, 0.0)
        t = _dot16(r, w2_ref[...])
        t16_s[pl.ds(m * _BM, _BM), :] = t.astype(jnp.bfloat16)

    @pl.when(p == 1)
    def _phase1():
        a16 = adj_ref[...].astype(jnp.bfloat16)
        h2 = jnp.dot(a16, t16_s[...],
                     preferred_element_type=jnp.float32) + b2_ref[...]
        y = _dot16(h2, lw1_ref[...]) + lb1_ref[...]
        x_ref[...] = _dot16(y, lw2_ref[...]) + lb2_ref[...]


def _gcn_pallas(adj, emb, w1, b1, w2, lw1, lb1, lw2, lb2, b2):
    n = adj.shape[0]
    num_m = n // _BM
    grid = (2, num_m)
    full = lambda shape: pl.BlockSpec(shape, lambda p, m: (0, 0))
    return pl.pallas_call(
        _gcn_body,
        grid=grid,
        in_specs=[
            pl.BlockSpec((_BM, n), lambda p, m: (m, 0)),
            full((n, NEMB)),
            full((NEMB, NEMB)),
            full((1, NEMB)),
            full((NEMB, NEMB)),
            full((NEMB, 16)),
            full((1, 16)),
            full((16, 1)),
            full((1, 1)),
            full((1, NEMB)),
        ],
        out_specs=pl.BlockSpec(
            (_BM, 1), lambda p, m: (jnp.where(p == 0, 0, m + 1), 0)),
        out_shape=jax.ShapeDtypeStruct((n + _BM, 1), jnp.float32),
        scratch_shapes=[
            pltpu.VMEM((n, NEMB), jnp.bfloat16),
            pltpu.VMEM((n, NEMB), jnp.bfloat16),
        ],
        compiler_params=pltpu.CompilerParams(
            dimension_semantics=("arbitrary", "arbitrary")),
    )(adj, emb, w1, b1, w2, lw1, lb1, lw2, lb2, b2)


def _sc_gather(emb_table, idx):
    return _make_sc_gather(emb_table.shape[0])(emb_table, idx)


def kernel(features, adj, emb_table, W1, b1, W2, b2, lw1, lb1, lw2, lb2):
    feats = features.astype(jnp.int32)
    user_emb = _sc_gather(emb_table, feats)
    x = _gcn_pallas(adj, user_emb, W1, b1.reshape(1, -1), W2, lw1,
                    lb1.reshape(1, -1), lw2, lb2.reshape(1, 1),
                    b2.reshape(1, -1))[_BM:]
    return (x, user_emb)
